# baseline (speedup 1.0000x reference)
import functools

import numpy as np
import jax
import jax.numpy as jnp
from jax import lax
from jax.experimental import pallas as pl
from jax.experimental.pallas import tpu as pltpu
from jax.experimental.pallas import tpu_sc as plsc

N = 10000
NP = 10240
E = 320000
NS = 16
NC = 2
CHUNK = 64
WIN = 320
SLAB = 16
NSLAB = WIN // SLAB
NBUF = 5
AHEAD = 3
EP = NS * WIN * CHUNK
STRIPE = NP // NS
HALF = 128
ROWB = 512

_mesh = plsc.VectorSubcoreMesh(core_axis_name="c", subcore_axis_name="s")


@functools.partial(
    pl.kernel,
    out_type=(jax.ShapeDtypeStruct((NP, HALF), jnp.float32),
              jax.ShapeDtypeStruct((NP, HALF), jnp.float32)),
    mesh=_mesh,
    scratch_types=(
        [pltpu.VMEM_SHARED((NP, HALF), jnp.float32)]
        + [pltpu.VMEM((SLAB, CHUNK), jnp.int32)] * 2
        + [pltpu.VMEM((CHUNK, HALF), jnp.float32)] * NBUF
        + [pltpu.SemaphoreType.DMA] * (2 * NBUF)
    ),
)
def _propagate(g0, g1, srcw, dstw, out0, out1,
               acc, sblk, dblk, r0, r1, r2, r3, r4,
               g0s, g1s, g2s, g3s, g4s, s0s, s1s, s2s, s3s, s4s):
    rows = (r0, r1, r2, r3, r4)
    gsem = (g0s, g1s, g2s, g3s, g4s)
    ssem = (s0s, s1s, s2s, s3s, s4s)
    s = lax.axis_index("s")
    c = lax.axis_index("c")
    stripe = pl.ds(s * STRIPE, STRIPE)

    def run(g, out):
        pltpu.sync_copy(g.at[stripe], acc.at[stripe])
        plsc.subcore_barrier()

        def scat_wait(b):
            pltpu.make_async_copy(rows[b], acc.at[dblk.at[b]], ssem[b]).wait()

        def slab(j, carry):
            @pl.when(j > 0)
            def _():
                for b in range(NBUF):
                    scat_wait(b)

            pltpu.sync_copy(srcw.at[s, pl.ds(j * SLAB, SLAB)], sblk)
            pltpu.sync_copy(dstw.at[s, pl.ds(j * SLAB, SLAB)], dblk)
            for b in range(AHEAD):
                pltpu.async_copy(g.at[sblk.at[b]], rows[b], gsem[b])
            for i in range(SLAB):
                bi = i % NBUF
                pltpu.make_async_copy(g.at[sblk.at[i]], rows[bi], gsem[bi]).wait()
                pltpu.async_copy(rows[bi], acc.at[dblk.at[i]], ssem[bi], add=True)
                if i + AHEAD < SLAB:
                    bn = (i + AHEAD) % NBUF
                    if i >= NBUF - AHEAD:
                        scat_wait(bn)
                    pltpu.async_copy(g.at[sblk.at[i + AHEAD]], rows[bn], gsem[bn])
            return carry

        lax.fori_loop(0, NSLAB, slab, 0)
        for b in range(NBUF):
            scat_wait(b)
        plsc.subcore_barrier()
        pltpu.sync_copy(acc.at[stripe], out.at[stripe])

    @pl.when(c == 0)
    def _():
        run(g0, out0)

    @pl.when(c == 1)
    def _():
        run(g1, out1)


def _dinv(d_ref):
    return lax.rsqrt(d_ref[:, 0:1])


def _lin1_body(x_ref, w_ref, d_ref, g0_ref, g1_ref):
    dinv = _dinv(d_ref)
    y = jnp.dot(x_ref[...], w_ref[...], preferred_element_type=jnp.float32)
    y = y * dinv
    g0_ref[...] = y[:, :HALF]
    g1_ref[...] = y[:, HALF:]


_lin1 = pl.pallas_call(
    _lin1_body,
    grid=(NP // ROWB,),
    in_specs=[
        pl.BlockSpec((ROWB, 128), lambda i: (i, 0)),
        pl.BlockSpec((128, 256), lambda i: (0, 0)),
        pl.BlockSpec((ROWB, HALF), lambda i: (i, 0)),
    ],
    out_specs=(pl.BlockSpec((ROWB, HALF), lambda i: (i, 0)),
               pl.BlockSpec((ROWB, HALF), lambda i: (i, 0))),
    out_shape=(jax.ShapeDtypeStruct((NP, HALF), jnp.float32),
               jax.ShapeDtypeStruct((NP, HALF), jnp.float32)),
)


def _mid_body(s0_ref, s1_ref, b_ref, w_ref, d_ref, g0_ref, g1_ref):
    dinv = _dinv(d_ref)
    h0 = jnp.maximum(s0_ref[...] * dinv + b_ref[0:1, :HALF], 0.0)
    h1 = jnp.maximum(s1_ref[...] * dinv + b_ref[0:1, HALF:], 0.0)
    h = jnp.concatenate([h0, h1], axis=1)
    y = jnp.dot(h, w_ref[...], preferred_element_type=jnp.float32)
    y = y * dinv
    g0_ref[...] = y[:, :HALF]
    g1_ref[...] = y[:, HALF:]


_mid = pl.pallas_call(
    _mid_body,
    grid=(NP // ROWB,),
    in_specs=[
        pl.BlockSpec((ROWB, HALF), lambda i: (i, 0)),
        pl.BlockSpec((ROWB, HALF), lambda i: (i, 0)),
        pl.BlockSpec((1, 256), lambda i: (0, 0)),
        pl.BlockSpec((256, 256), lambda i: (0, 0)),
        pl.BlockSpec((ROWB, HALF), lambda i: (i, 0)),
    ],
    out_specs=(pl.BlockSpec((ROWB, HALF), lambda i: (i, 0)),
               pl.BlockSpec((ROWB, HALF), lambda i: (i, 0))),
    out_shape=(jax.ShapeDtypeStruct((NP, HALF), jnp.float32),
               jax.ShapeDtypeStruct((NP, HALF), jnp.float32)),
)


def _out_body(s0_ref, s1_ref, b_ref, w_ref, bo_ref, d_ref, o_ref):
    dinv = _dinv(d_ref)
    h0 = jnp.maximum(s0_ref[...] * dinv + b_ref[0:1, :HALF], 0.0)
    h1 = jnp.maximum(s1_ref[...] * dinv + b_ref[0:1, HALF:], 0.0)
    h = jnp.concatenate([h0, h1], axis=1)
    o_ref[...] = (jnp.dot(h, w_ref[...], preferred_element_type=jnp.float32)
                  + bo_ref[0:1, :])


_outk = pl.pallas_call(
    _out_body,
    grid=(NP // ROWB,),
    in_specs=[
        pl.BlockSpec((ROWB, HALF), lambda i: (i, 0)),
        pl.BlockSpec((ROWB, HALF), lambda i: (i, 0)),
        pl.BlockSpec((1, 256), lambda i: (0, 0)),
        pl.BlockSpec((256, 128), lambda i: (0, 0)),
        pl.BlockSpec((1, 128), lambda i: (0, 0)),
        pl.BlockSpec((ROWB, HALF), lambda i: (i, 0)),
    ],
    out_specs=pl.BlockSpec((ROWB, 128), lambda i: (i, 0)),
    out_shape=jax.ShapeDtypeStruct((NP, 128), jnp.float32),
)


def kernel(x, edge_index, W1, b1, W2, b2, W3, b3, Wo, bo):
    src = edge_index[0].astype(jnp.int32)
    dst = edge_index[1].astype(jnp.int32)
    npad = EP - E
    pad_src = jnp.asarray((np.arange(npad) * 7919) % N, dtype=jnp.int32)
    pad_dst = jnp.asarray(N + (np.arange(npad) % (NP - N)), dtype=jnp.int32)
    srcw = jnp.concatenate([src, pad_src]).reshape(NS, WIN, CHUNK)
    dstw = jnp.concatenate([dst, pad_dst]).reshape(NS, WIN, CHUNK)
    xp = jnp.pad(x, ((0, NP - N), (0, 0)))

    ones_g = jnp.ones((NP, HALF), jnp.float32)
    deg, _unused = _propagate(ones_g, ones_g, srcw, dstw)
    b1r = b1.reshape(1, 256)
    b2r = b2.reshape(1, 256)
    b3r = b3.reshape(1, 256)
    bor = bo.reshape(1, 128)

    g0, g1 = _lin1(xp, W1, deg)
    s0, s1 = _propagate(g0, g1, srcw, dstw)
    g0, g1 = _mid(s0, s1, b1r, W2, deg)
    s0, s1 = _propagate(g0, g1, srcw, dstw)
    g0, g1 = _mid(s0, s1, b2r, W3, deg)
    s0, s1 = _propagate(g0, g1, srcw, dstw)
    out = _outk(s0, s1, b3r, Wo, bor, deg)
    return out[:N]


# speedup vs baseline: 1.1585x; 1.1585x over previous
import functools

import numpy as np
import jax
import jax.numpy as jnp
from jax import lax
from jax.experimental import pallas as pl
from jax.experimental.pallas import tpu as pltpu
from jax.experimental.pallas import tpu_sc as plsc

N = 10000
NP = 10240
E = 320000
NS = 16
NC = 2
CHUNK = 64
WIN = 320
SLAB = 16
NSLAB = WIN // SLAB
NBUF = 5
AHEAD = 3
DCH = 128
DSLAB = 8
DWIN = WIN // 2
NSLABD = DWIN // DSLAB
EP = NS * WIN * CHUNK
STRIPE = NP // NS
HALF = 128
ROWB = 512

_mesh = plsc.VectorSubcoreMesh(core_axis_name="c", subcore_axis_name="s")


@functools.partial(
    pl.kernel,
    out_type=(jax.ShapeDtypeStruct((NP, HALF), jnp.float32),
              jax.ShapeDtypeStruct((NP, HALF), jnp.float32)),
    mesh=_mesh,
    scratch_types=[
        pltpu.VMEM_SHARED((NP, HALF), jnp.float32),
        pltpu.VMEM((DSLAB, DCH), jnp.int32),
        pltpu.VMEM((DCH, HALF), jnp.float32),
    ],
)
def _degree(dstw_d, ones_g, ones_h, deg0, deg1, acc, dblk, onesv):
    s = lax.axis_index("s")
    c = lax.axis_index("c")
    stripe = pl.ds(s * STRIPE, STRIPE)
    pltpu.sync_copy(ones_h, onesv)
    pltpu.sync_copy(ones_g.at[stripe], acc.at[stripe])
    plsc.subcore_barrier()

    def slab(t, carry):
        j = 2 * t + c
        pltpu.sync_copy(dstw_d.at[s, pl.ds(j * DSLAB, DSLAB)], dblk)
        for i in range(DSLAB):
            pltpu.sync_copy(onesv, acc.at[dblk.at[i]], add=True)
        return carry

    lax.fori_loop(0, NSLABD // 2, slab, 0)
    plsc.subcore_barrier()

    @pl.when(c == 0)
    def _():
        pltpu.sync_copy(acc.at[stripe], deg0.at[stripe])

    @pl.when(c == 1)
    def _():
        pltpu.sync_copy(acc.at[stripe], deg1.at[stripe])


@functools.partial(
    pl.kernel,
    out_type=(jax.ShapeDtypeStruct((NP, HALF), jnp.float32),
              jax.ShapeDtypeStruct((NP, HALF), jnp.float32)),
    mesh=_mesh,
    scratch_types=(
        [pltpu.VMEM_SHARED((NP, HALF), jnp.float32)]
        + [pltpu.VMEM((SLAB, CHUNK), jnp.int32)] * 2
        + [pltpu.VMEM((CHUNK, HALF), jnp.float32)] * NBUF
        + [pltpu.SemaphoreType.DMA] * (2 * NBUF)
    ),
)
def _propagate(g0, g1, srcw, dstw, out0, out1,
               acc, sblk, dblk, r0, r1, r2, r3, r4,
               g0s, g1s, g2s, g3s, g4s, s0s, s1s, s2s, s3s, s4s):
    rows = (r0, r1, r2, r3, r4)
    gsem = (g0s, g1s, g2s, g3s, g4s)
    ssem = (s0s, s1s, s2s, s3s, s4s)
    s = lax.axis_index("s")
    c = lax.axis_index("c")
    stripe = pl.ds(s * STRIPE, STRIPE)

    def run(g, out):
        pltpu.sync_copy(g.at[stripe], acc.at[stripe])
        plsc.subcore_barrier()

        def scat_wait(b):
            pltpu.make_async_copy(rows[b], acc.at[dblk.at[b]], ssem[b]).wait()

        def slab(j, carry):
            @pl.when(j > 0)
            def _():
                for b in range(NBUF):
                    scat_wait(b)

            pltpu.sync_copy(srcw.at[s, pl.ds(j * SLAB, SLAB)], sblk)
            pltpu.sync_copy(dstw.at[s, pl.ds(j * SLAB, SLAB)], dblk)
            for b in range(AHEAD):
                pltpu.async_copy(g.at[sblk.at[b]], rows[b], gsem[b])
            for i in range(SLAB):
                bi = i % NBUF
                pltpu.make_async_copy(g.at[sblk.at[i]], rows[bi], gsem[bi]).wait()
                pltpu.async_copy(rows[bi], acc.at[dblk.at[i]], ssem[bi], add=True)
                if i + AHEAD < SLAB:
                    bn = (i + AHEAD) % NBUF
                    if i >= NBUF - AHEAD:
                        scat_wait(bn)
                    pltpu.async_copy(g.at[sblk.at[i + AHEAD]], rows[bn], gsem[bn])
            return carry

        lax.fori_loop(0, NSLAB, slab, 0)
        for b in range(NBUF):
            scat_wait(b)
        plsc.subcore_barrier()
        pltpu.sync_copy(acc.at[stripe], out.at[stripe])

    @pl.when(c == 0)
    def _():
        run(g0, out0)

    @pl.when(c == 1)
    def _():
        run(g1, out1)


def _dinv(d0_ref, d1_ref):
    return lax.rsqrt(d0_ref[:, 0:1] + d1_ref[:, 0:1] - 1.0)


def _lin1_body(x_ref, w_ref, d0_ref, d1_ref, g0_ref, g1_ref):
    dinv = _dinv(d0_ref, d1_ref)
    y = jnp.dot(x_ref[...], w_ref[...], preferred_element_type=jnp.float32)
    y = y * dinv
    g0_ref[...] = y[:, :HALF]
    g1_ref[...] = y[:, HALF:]


_lin1 = pl.pallas_call(
    _lin1_body,
    grid=(NP // ROWB,),
    in_specs=[
        pl.BlockSpec((ROWB, 128), lambda i: (i, 0)),
        pl.BlockSpec((128, 256), lambda i: (0, 0)),
        pl.BlockSpec((ROWB, HALF), lambda i: (i, 0)),
        pl.BlockSpec((ROWB, HALF), lambda i: (i, 0)),
    ],
    out_specs=(pl.BlockSpec((ROWB, HALF), lambda i: (i, 0)),
               pl.BlockSpec((ROWB, HALF), lambda i: (i, 0))),
    out_shape=(jax.ShapeDtypeStruct((NP, HALF), jnp.float32),
               jax.ShapeDtypeStruct((NP, HALF), jnp.float32)),
)


def _mid_body(s0_ref, s1_ref, b_ref, w_ref, d0_ref, d1_ref, g0_ref, g1_ref):
    dinv = _dinv(d0_ref, d1_ref)
    h0 = jnp.maximum(s0_ref[...] * dinv + b_ref[0:1, :HALF], 0.0)
    h1 = jnp.maximum(s1_ref[...] * dinv + b_ref[0:1, HALF:], 0.0)
    h = jnp.concatenate([h0, h1], axis=1)
    y = jnp.dot(h, w_ref[...], preferred_element_type=jnp.float32)
    y = y * dinv
    g0_ref[...] = y[:, :HALF]
    g1_ref[...] = y[:, HALF:]


_mid = pl.pallas_call(
    _mid_body,
    grid=(NP // ROWB,),
    in_specs=[
        pl.BlockSpec((ROWB, HALF), lambda i: (i, 0)),
        pl.BlockSpec((ROWB, HALF), lambda i: (i, 0)),
        pl.BlockSpec((1, 256), lambda i: (0, 0)),
        pl.BlockSpec((256, 256), lambda i: (0, 0)),
        pl.BlockSpec((ROWB, HALF), lambda i: (i, 0)),
        pl.BlockSpec((ROWB, HALF), lambda i: (i, 0)),
    ],
    out_specs=(pl.BlockSpec((ROWB, HALF), lambda i: (i, 0)),
               pl.BlockSpec((ROWB, HALF), lambda i: (i, 0))),
    out_shape=(jax.ShapeDtypeStruct((NP, HALF), jnp.float32),
               jax.ShapeDtypeStruct((NP, HALF), jnp.float32)),
)


def _out_body(s0_ref, s1_ref, b_ref, w_ref, bo_ref, d0_ref, d1_ref, o_ref):
    dinv = _dinv(d0_ref, d1_ref)
    h0 = jnp.maximum(s0_ref[...] * dinv + b_ref[0:1, :HALF], 0.0)
    h1 = jnp.maximum(s1_ref[...] * dinv + b_ref[0:1, HALF:], 0.0)
    h = jnp.concatenate([h0, h1], axis=1)
    o_ref[...] = (jnp.dot(h, w_ref[...], preferred_element_type=jnp.float32)
                  + bo_ref[0:1, :])


_outk = pl.pallas_call(
    _out_body,
    grid=(NP // ROWB,),
    in_specs=[
        pl.BlockSpec((ROWB, HALF), lambda i: (i, 0)),
        pl.BlockSpec((ROWB, HALF), lambda i: (i, 0)),
        pl.BlockSpec((1, 256), lambda i: (0, 0)),
        pl.BlockSpec((256, 128), lambda i: (0, 0)),
        pl.BlockSpec((1, 128), lambda i: (0, 0)),
        pl.BlockSpec((ROWB, HALF), lambda i: (i, 0)),
        pl.BlockSpec((ROWB, HALF), lambda i: (i, 0)),
    ],
    out_specs=pl.BlockSpec((ROWB, 128), lambda i: (i, 0)),
    out_shape=jax.ShapeDtypeStruct((NP, 128), jnp.float32),
)


def kernel(x, edge_index, W1, b1, W2, b2, W3, b3, Wo, bo):
    src = edge_index[0].astype(jnp.int32)
    dst = edge_index[1].astype(jnp.int32)
    npad = EP - E
    pad_src = jnp.asarray((np.arange(npad) * 7919) % N, dtype=jnp.int32)
    pad_dst = jnp.asarray(N + (np.arange(npad) % (NP - N)), dtype=jnp.int32)
    srcw = jnp.concatenate([src, pad_src]).reshape(NS, WIN, CHUNK)
    dstw = jnp.concatenate([dst, pad_dst]).reshape(NS, WIN, CHUNK)
    xp = jnp.pad(x, ((0, NP - N), (0, 0)))

    dstw_d = dstw.reshape(NS, DWIN, DCH)
    ones_g = jnp.ones((NP, HALF), jnp.float32)
    ones_h = jnp.ones((DCH, HALF), jnp.float32)
    d0, d1 = _degree(dstw_d, ones_g, ones_h)
    b1r = b1.reshape(1, 256)
    b2r = b2.reshape(1, 256)
    b3r = b3.reshape(1, 256)
    bor = bo.reshape(1, 128)

    g0, g1 = _lin1(xp, W1, d0, d1)
    s0, s1 = _propagate(g0, g1, srcw, dstw)
    g0, g1 = _mid(s0, s1, b1r, W2, d0, d1)
    s0, s1 = _propagate(g0, g1, srcw, dstw)
    g0, g1 = _mid(s0, s1, b2r, W3, d0, d1)
    s0, s1 = _propagate(g0, g1, srcw, dstw)
    out = _outk(s0, s1, b3r, Wo, bor, d0, d1)
    return out[:N]


# speedup vs baseline: 1.1695x; 1.0095x over previous
import functools

import numpy as np
import jax
import jax.numpy as jnp
from jax import lax
from jax.experimental import pallas as pl
from jax.experimental.pallas import tpu as pltpu
from jax.experimental.pallas import tpu_sc as plsc

N = 10000
NP = 10240
E = 320000
NS = 16
NC = 2
CHUNK = 64
WIN = 320
SLAB = 16
NSLAB = WIN // SLAB
NBUF = 5
AHEAD = 4
DCH = 128
DSLAB = 8
DWIN = WIN // 2
NSLABD = DWIN // DSLAB
EP = NS * WIN * CHUNK
STRIPE = NP // NS
HALF = 128
ROWB = 512

_mesh = plsc.VectorSubcoreMesh(core_axis_name="c", subcore_axis_name="s")


@functools.partial(
    pl.kernel,
    out_type=(jax.ShapeDtypeStruct((NP, HALF), jnp.float32),
              jax.ShapeDtypeStruct((NP, HALF), jnp.float32)),
    mesh=_mesh,
    scratch_types=[
        pltpu.VMEM_SHARED((NP, HALF), jnp.float32),
        pltpu.VMEM((DSLAB, DCH), jnp.int32),
        pltpu.VMEM((DCH, HALF), jnp.float32),
    ],
)
def _degree(dstw_d, ones_g, ones_h, deg0, deg1, acc, dblk, onesv):
    s = lax.axis_index("s")
    c = lax.axis_index("c")
    stripe = pl.ds(s * STRIPE, STRIPE)
    pltpu.sync_copy(ones_h, onesv)
    pltpu.sync_copy(ones_g.at[stripe], acc.at[stripe])
    plsc.subcore_barrier()

    def slab(t, carry):
        j = 2 * t + c
        pltpu.sync_copy(dstw_d.at[s, pl.ds(j * DSLAB, DSLAB)], dblk)
        for i in range(DSLAB):
            pltpu.sync_copy(onesv, acc.at[dblk.at[i]], add=True)
        return carry

    lax.fori_loop(0, NSLABD // 2, slab, 0)
    plsc.subcore_barrier()

    @pl.when(c == 0)
    def _():
        pltpu.sync_copy(acc.at[stripe], deg0.at[stripe])

    @pl.when(c == 1)
    def _():
        pltpu.sync_copy(acc.at[stripe], deg1.at[stripe])


@functools.partial(
    pl.kernel,
    out_type=(jax.ShapeDtypeStruct((NP, HALF), jnp.float32),
              jax.ShapeDtypeStruct((NP, HALF), jnp.float32)),
    mesh=_mesh,
    scratch_types=(
        [pltpu.VMEM_SHARED((NP, HALF), jnp.float32)]
        + [pltpu.VMEM((SLAB, CHUNK), jnp.int32)] * 2
        + [pltpu.VMEM((CHUNK, HALF), jnp.float32)] * NBUF
        + [pltpu.SemaphoreType.DMA] * (2 * NBUF)
    ),
)
def _propagate(g0, g1, srcw, dstw, out0, out1,
               acc, sblk, dblk, r0, r1, r2, r3, r4,
               g0s, g1s, g2s, g3s, g4s, s0s, s1s, s2s, s3s, s4s):
    rows = (r0, r1, r2, r3, r4)
    gsem = (g0s, g1s, g2s, g3s, g4s)
    ssem = (s0s, s1s, s2s, s3s, s4s)
    s = lax.axis_index("s")
    c = lax.axis_index("c")
    stripe = pl.ds(s * STRIPE, STRIPE)

    def run(g, out):
        pltpu.sync_copy(g.at[stripe], acc.at[stripe])
        plsc.subcore_barrier()

        def scat_wait(b):
            pltpu.make_async_copy(rows[b], acc.at[dblk.at[b]], ssem[b]).wait()

        def slab(j, carry):
            @pl.when(j > 0)
            def _():
                for b in range(NBUF):
                    scat_wait(b)

            pltpu.sync_copy(srcw.at[s, pl.ds(j * SLAB, SLAB)], sblk)
            pltpu.sync_copy(dstw.at[s, pl.ds(j * SLAB, SLAB)], dblk)
            for b in range(AHEAD):
                pltpu.async_copy(g.at[sblk.at[b]], rows[b], gsem[b])
            for i in range(SLAB):
                bi = i % NBUF
                pltpu.make_async_copy(g.at[sblk.at[i]], rows[bi], gsem[bi]).wait()
                pltpu.async_copy(rows[bi], acc.at[dblk.at[i]], ssem[bi], add=True)
                if i + AHEAD < SLAB:
                    bn = (i + AHEAD) % NBUF
                    if i >= NBUF - AHEAD:
                        scat_wait(bn)
                    pltpu.async_copy(g.at[sblk.at[i + AHEAD]], rows[bn], gsem[bn])
            return carry

        lax.fori_loop(0, NSLAB, slab, 0)
        for b in range(NBUF):
            scat_wait(b)
        plsc.subcore_barrier()
        pltpu.sync_copy(acc.at[stripe], out.at[stripe])

    @pl.when(c == 0)
    def _():
        run(g0, out0)

    @pl.when(c == 1)
    def _():
        run(g1, out1)


def _dinv(d0_ref, d1_ref):
    return lax.rsqrt(d0_ref[:, 0:1] + d1_ref[:, 0:1] - 1.0)


def _lin1_body(x_ref, w_ref, d0_ref, d1_ref, g0_ref, g1_ref):
    dinv = _dinv(d0_ref, d1_ref)
    y = jnp.dot(x_ref[...], w_ref[...], preferred_element_type=jnp.float32)
    y = y * dinv
    g0_ref[...] = y[:, :HALF]
    g1_ref[...] = y[:, HALF:]


_lin1 = pl.pallas_call(
    _lin1_body,
    grid=(NP // ROWB,),
    in_specs=[
        pl.BlockSpec((ROWB, 128), lambda i: (i, 0)),
        pl.BlockSpec((128, 256), lambda i: (0, 0)),
        pl.BlockSpec((ROWB, HALF), lambda i: (i, 0)),
        pl.BlockSpec((ROWB, HALF), lambda i: (i, 0)),
    ],
    out_specs=(pl.BlockSpec((ROWB, HALF), lambda i: (i, 0)),
               pl.BlockSpec((ROWB, HALF), lambda i: (i, 0))),
    out_shape=(jax.ShapeDtypeStruct((NP, HALF), jnp.float32),
               jax.ShapeDtypeStruct((NP, HALF), jnp.float32)),
)


def _mid_body(s0_ref, s1_ref, b_ref, w_ref, d0_ref, d1_ref, g0_ref, g1_ref):
    dinv = _dinv(d0_ref, d1_ref)
    h0 = jnp.maximum(s0_ref[...] * dinv + b_ref[0:1, :HALF], 0.0)
    h1 = jnp.maximum(s1_ref[...] * dinv + b_ref[0:1, HALF:], 0.0)
    h = jnp.concatenate([h0, h1], axis=1)
    y = jnp.dot(h, w_ref[...], preferred_element_type=jnp.float32)
    y = y * dinv
    g0_ref[...] = y[:, :HALF]
    g1_ref[...] = y[:, HALF:]


_mid = pl.pallas_call(
    _mid_body,
    grid=(NP // ROWB,),
    in_specs=[
        pl.BlockSpec((ROWB, HALF), lambda i: (i, 0)),
        pl.BlockSpec((ROWB, HALF), lambda i: (i, 0)),
        pl.BlockSpec((1, 256), lambda i: (0, 0)),
        pl.BlockSpec((256, 256), lambda i: (0, 0)),
        pl.BlockSpec((ROWB, HALF), lambda i: (i, 0)),
        pl.BlockSpec((ROWB, HALF), lambda i: (i, 0)),
    ],
    out_specs=(pl.BlockSpec((ROWB, HALF), lambda i: (i, 0)),
               pl.BlockSpec((ROWB, HALF), lambda i: (i, 0))),
    out_shape=(jax.ShapeDtypeStruct((NP, HALF), jnp.float32),
               jax.ShapeDtypeStruct((NP, HALF), jnp.float32)),
)


def _out_body(s0_ref, s1_ref, b_ref, w_ref, bo_ref, d0_ref, d1_ref, o_ref):
    dinv = _dinv(d0_ref, d1_ref)
    h0 = jnp.maximum(s0_ref[...] * dinv + b_ref[0:1, :HALF], 0.0)
    h1 = jnp.maximum(s1_ref[...] * dinv + b_ref[0:1, HALF:], 0.0)
    h = jnp.concatenate([h0, h1], axis=1)
    o_ref[...] = (jnp.dot(h, w_ref[...], preferred_element_type=jnp.float32)
                  + bo_ref[0:1, :])


_outk = pl.pallas_call(
    _out_body,
    grid=(NP // ROWB,),
    in_specs=[
        pl.BlockSpec((ROWB, HALF), lambda i: (i, 0)),
        pl.BlockSpec((ROWB, HALF), lambda i: (i, 0)),
        pl.BlockSpec((1, 256), lambda i: (0, 0)),
        pl.BlockSpec((256, 128), lambda i: (0, 0)),
        pl.BlockSpec((1, 128), lambda i: (0, 0)),
        pl.BlockSpec((ROWB, HALF), lambda i: (i, 0)),
        pl.BlockSpec((ROWB, HALF), lambda i: (i, 0)),
    ],
    out_specs=pl.BlockSpec((ROWB, 128), lambda i: (i, 0)),
    out_shape=jax.ShapeDtypeStruct((NP, 128), jnp.float32),
)


def kernel(x, edge_index, W1, b1, W2, b2, W3, b3, Wo, bo):
    src = edge_index[0].astype(jnp.int32)
    dst = edge_index[1].astype(jnp.int32)
    npad = EP - E
    pad_src = jnp.asarray((np.arange(npad) * 7919) % N, dtype=jnp.int32)
    pad_dst = jnp.asarray(N + (np.arange(npad) % (NP - N)), dtype=jnp.int32)
    srcw = jnp.concatenate([src, pad_src]).reshape(NS, WIN, CHUNK)
    dstw = jnp.concatenate([dst, pad_dst]).reshape(NS, WIN, CHUNK)
    xp = jnp.pad(x, ((0, NP - N), (0, 0)))

    dstw_d = dstw.reshape(NS, DWIN, DCH)
    ones_g = jnp.ones((NP, HALF), jnp.float32)
    ones_h = jnp.ones((DCH, HALF), jnp.float32)
    d0, d1 = _degree(dstw_d, ones_g, ones_h)
    b1r = b1.reshape(1, 256)
    b2r = b2.reshape(1, 256)
    b3r = b3.reshape(1, 256)
    bor = bo.reshape(1, 128)

    g0, g1 = _lin1(xp, W1, d0, d1)
    s0, s1 = _propagate(g0, g1, srcw, dstw)
    g0, g1 = _mid(s0, s1, b1r, W2, d0, d1)
    s0, s1 = _propagate(g0, g1, srcw, dstw)
    g0, g1 = _mid(s0, s1, b2r, W3, d0, d1)
    s0, s1 = _propagate(g0, g1, srcw, dstw)
    out = _outk(s0, s1, b3r, Wo, bor, d0, d1)
    return out[:N]
